# baseline (speedup 1.0000x reference)
import functools

import jax
import jax.numpy as jnp
from jax import lax
from jax.experimental import pallas as pl
from jax.experimental.pallas import tpu as pltpu
from jax.experimental.pallas import tpu_sc as plsc

E = 8
SHARED = 1
TOPK = 2
D = 1024
DFF = 2048
NR = E - SHARED

T = 2048
M = 256
NTS = T // M
NTR = (TOPK * T) // M + (NR - 1)
NT = NTS + NTR
NTPAD = 32
LPR = NTR * M
NP = TOPK * T


def _router_body(x_ref, rw_ref, dst_ref, wv_ref, meta_ref):
    logits = lax.dot_general(
        x_ref[...], rw_ref[...], (((1,), (1,)), ((), ())),
        preferred_element_type=jnp.float32)
    m = jnp.max(logits, axis=-1, keepdims=True)
    ex = jnp.exp(logits - m)
    probs = ex / jnp.sum(ex, axis=-1, keepdims=True)

    col = lax.broadcasted_iota(jnp.int32, probs.shape, 1)
    v1 = jnp.max(probs, axis=-1, keepdims=True)
    i1 = jnp.min(jnp.where(probs == v1, col, NR), axis=-1, keepdims=True)
    pm = jnp.where(col == i1, -jnp.inf, probs)
    v2 = jnp.max(pm, axis=-1, keepdims=True)
    i2 = jnp.min(jnp.where(pm == v2, col, NR), axis=-1, keepdims=True)
    denom = v1 + v2 + 1e-9
    wv_ref[...] = jnp.concatenate([v1 / denom, v2 / denom], axis=1)

    a0 = (col == i1).astype(jnp.float32)
    a1 = (col == i2).astype(jnp.float32)
    s = a0 + a1
    r_io = lax.broadcasted_iota(jnp.int32, (T, T), 0)
    c_io = lax.broadcasted_iota(jnp.int32, (T, T), 1)
    tri = (c_io < r_io).astype(jnp.float32)
    xc = lax.dot_general(tri, s, (((1,), (0,)), ((), ())),
                         preferred_element_type=jnp.float32)
    counts = jnp.sum(s, axis=0, keepdims=True)

    nt = jnp.floor((counts + (M - 1)) * (1.0 / M))
    e_r = lax.broadcasted_iota(jnp.int32, (NR, NR), 0)
    e_c = lax.broadcasted_iota(jnp.int32, (NR, NR), 1)
    ltri = (e_c <= e_r).astype(jnp.float32)
    cumi = lax.dot_general(nt, ltri, (((1,), (1,)), ((), ())),
                           preferred_element_type=jnp.float32)
    cume = cumi - nt
    pb = M * cume
    dst0 = jnp.sum(a0 * (pb + xc), axis=1, keepdims=True)
    dst1 = jnp.sum(a1 * (pb + xc), axis=1, keepdims=True)
    dst_ref[...] = jnp.concatenate([dst0, dst1], axis=1).astype(jnp.int32)

    j_io = lax.broadcasted_iota(jnp.int32, (NTPAD, NR), 0)
    total_i = jnp.sum(nt, axis=1, keepdims=True).astype(jnp.int32)
    cumi_i = cumi.astype(jnp.int32)
    j_col = lax.broadcasted_iota(jnp.int32, (NTPAD, 1), 0)
    te_full = jnp.minimum(
        jnp.sum((j_io >= cumi_i).astype(jnp.int32), axis=1,
                keepdims=True) + SHARED, E - 1)
    tv_full = (j_col < total_i).astype(jnp.int32)
    meta_ref[...] = jnp.concatenate([te_full, tv_full], axis=1)


def _router(flat, router_w):
    return pl.pallas_call(
        _router_body,
        out_shape=[
            jax.ShapeDtypeStruct((T, 2), jnp.int32),
            jax.ShapeDtypeStruct((T, 2), jnp.float32),
            jax.ShapeDtypeStruct((NTPAD, 2), jnp.int32),
        ],
    )(flat, router_w)


def _sc_info():
    info = plsc.get_sparse_core_info()
    return info.num_cores, info.num_subcores


def _scatter_kernel_fn(dst_hbm, wv_hbm, tid_hbm, sw_hbm,
                       dst_v, wv_v, tid_v, sw_v):
    nc, _ = _sc_info()
    wid = lax.axis_index("s") * nc + lax.axis_index("c")

    @pl.when(wid == 0)
    def _():
        pltpu.sync_copy(dst_hbm, dst_v)
        pltpu.sync_copy(wv_hbm, wv_v)
        lane = lax.iota(jnp.int32, 16)
        half = lax.shift_right_logical(lane, 1)
        zeros_i = jnp.zeros((16,), jnp.int32)
        zeros_f = jnp.zeros((16,), jnp.float32)

        def init_pad(i, carry):
            tid_v[pl.ds(i * 16, 16)] = zeros_i
            sw_v[pl.ds(i * 16, 16)] = zeros_f
            return carry

        lax.fori_loop(0, LPR // 16, init_pad, 0)

        def scat_body(i, carry):
            base = i * 16
            idx = dst_v[pl.ds(base, 16)]
            vals = wv_v[pl.ds(base, 16)]
            toks = half + i * 8
            plsc.store_scatter(tid_v, [idx], toks)
            plsc.store_scatter(sw_v, [idx], vals)
            return carry

        lax.fori_loop(0, NP // 16, scat_body, 0)
        pltpu.sync_copy(tid_v, tid_hbm)
        pltpu.sync_copy(sw_v, sw_hbm)


def _sc_scatter(dst, wv):
    mesh = plsc.VectorSubcoreMesh(core_axis_name="c", subcore_axis_name="s")
    k = functools.partial(
        pl.kernel, mesh=mesh,
        compiler_params=pltpu.CompilerParams(needs_layout_passes=False),
        out_type=[
            jax.ShapeDtypeStruct((LPR,), jnp.int32),
            jax.ShapeDtypeStruct((LPR,), jnp.float32),
        ],
        scratch_types=[
            pltpu.VMEM((NP,), jnp.int32),
            pltpu.VMEM((NP,), jnp.float32),
            pltpu.VMEM((LPR,), jnp.int32),
            pltpu.VMEM((LPR,), jnp.float32),
        ],
    )(_scatter_kernel_fn)
    return k(dst.reshape(NP), wv.reshape(NP))


def _row_gather_body(src_hbm, idx_hbm, out_hbm, idx_v, rows_a, rows_b, sem,
                     *, per_w, nch):
    nc, ns = _sc_info()
    wid = lax.axis_index("s") * nc + lax.axis_index("c")
    chunk = per_w // nch
    base = wid * per_w
    pltpu.sync_copy(idx_hbm.at[pl.ds(base, per_w)], idx_v)
    bufs = (rows_a, rows_b)
    cps = []
    for c in range(nch):
        if c >= 2:
            cps[c - 2].wait()
            pltpu.sync_copy(bufs[(c - 2) % 2],
                            out_hbm.at[pl.ds(base + (c - 2) * chunk, chunk)])
        cps.append(pltpu.async_copy(
            src_hbm.at[idx_v.at[pl.ds(c * chunk, chunk)]], bufs[c % 2], sem))
    for c in range(max(0, nch - 2), nch):
        cps[c].wait()
        pltpu.sync_copy(bufs[c % 2],
                        out_hbm.at[pl.ds(base + c * chunk, chunk)])


def _sc_row_gather(src, idx, n_out, nch):
    mesh = plsc.VectorSubcoreMesh(core_axis_name="c", subcore_axis_name="s")
    nc, ns = _sc_info()
    per_w = n_out // (nc * ns)
    chunk = per_w // nch
    body = functools.partial(_row_gather_body, per_w=per_w, nch=nch)
    k = functools.partial(
        pl.kernel, mesh=mesh,
        out_type=jax.ShapeDtypeStruct((n_out, D), jnp.float32),
        scratch_types=[
            pltpu.VMEM((per_w,), jnp.int32),
            pltpu.VMEM((chunk, D), jnp.float32),
            pltpu.VMEM((chunk, D), jnp.float32),
            pltpu.SemaphoreType.DMA,
        ],
    )(body)
    return k(src, idx)


def _shared_body(x_ref, up_ref, gate_ref, down_ref, out_ref):
    xb = x_ref[...].astype(jnp.bfloat16)
    u = lax.dot_general(xb, up_ref[...], (((1,), (1,)), ((), ())),
                        preferred_element_type=jnp.float32)
    g = lax.dot_general(xb, gate_ref[...], (((1,), (1,)), ((), ())),
                        preferred_element_type=jnp.float32)
    h = (g * jax.nn.sigmoid(g) * u).astype(jnp.bfloat16)
    out_ref[...] = lax.dot_general(h, down_ref[...], (((1,), (1,)), ((), ())),
                                   preferred_element_type=jnp.float32)


def _shared_ffn(flat, up0, gate0, down0):
    MS = 512
    return pl.pallas_call(
        _shared_body,
        grid=(T // MS,),
        in_specs=[
            pl.BlockSpec((MS, D), lambda t: (t, 0)),
            pl.BlockSpec((DFF, D), lambda t: (0, 0)),
            pl.BlockSpec((DFF, D), lambda t: (0, 0)),
            pl.BlockSpec((D, DFF), lambda t: (0, 0)),
        ],
        out_specs=pl.BlockSpec((MS, D), lambda t: (t, 0)),
        out_shape=jax.ShapeDtypeStruct((T, D), jnp.float32),
    )(flat, up0, gate0, down0)


def _gffn_body(te_ref, tv_ref, sw_ref, xs_ref, up_ref, gate_ref, down_ref,
               os_ref):
    i = pl.program_id(0)

    @pl.when(tv_ref[i] == 1)
    def _():
        xb = xs_ref[...].astype(jnp.bfloat16)
        u = lax.dot_general(xb, up_ref[0], (((1,), (1,)), ((), ())),
                            preferred_element_type=jnp.float32)
        g = lax.dot_general(xb, gate_ref[0], (((1,), (1,)), ((), ())),
                            preferred_element_type=jnp.float32)
        h = g * jax.nn.sigmoid(g) * u
        h = (h * sw_ref[0, 0][:, None]).astype(jnp.bfloat16)
        os_ref[...] = lax.dot_general(h, down_ref[0], (((1,), (1,)), ((), ())),
                                      preferred_element_type=jnp.float32)


def _grouped_ffn(te, tv, sw, xs, up, gate, down):
    grid_spec = pltpu.PrefetchScalarGridSpec(
        num_scalar_prefetch=2,
        grid=(NTR,),
        in_specs=[
            pl.BlockSpec((1, 1, M), lambda i, te, tv: (i, 0, 0)),
            pl.BlockSpec((M, D), lambda i, te, tv: (i, 0)),
            pl.BlockSpec((1, DFF, D), lambda i, te, tv: (te[i], 0, 0)),
            pl.BlockSpec((1, DFF, D), lambda i, te, tv: (te[i], 0, 0)),
            pl.BlockSpec((1, D, DFF), lambda i, te, tv: (te[i], 0, 0)),
        ],
        out_specs=pl.BlockSpec((M, D), lambda i, te, tv: (i, 0)),
    )
    return pl.pallas_call(
        _gffn_body,
        grid_spec=grid_spec,
        out_shape=jax.ShapeDtypeStruct((LPR, D), jnp.float32),
        compiler_params=pltpu.CompilerParams(
            dimension_semantics=("arbitrary",),
        ),
    )(te, tv, sw.reshape(NTR, 1, M), xs, up, gate, down)


def _combine_body(os_ref, yp_ref, out_ref):
    out_ref[...] = (os_ref[...] + yp_ref[:, :D] + yp_ref[:, D:])


def _combine(os, yp2):
    BT = 512
    return pl.pallas_call(
        _combine_body,
        grid=(T // BT,),
        in_specs=[
            pl.BlockSpec((BT, D), lambda t: (t, 0)),
            pl.BlockSpec((BT, 2 * D), lambda t: (t, 0)),
        ],
        out_specs=pl.BlockSpec((BT, D), lambda t: (t, 0)),
        out_shape=jax.ShapeDtypeStruct((T, D), jnp.float32),
    )(os, yp2)


def kernel(x, up, gate, down, router_w):
    orig_shape = x.shape
    flat = x.reshape(-1, D)
    upb = up.astype(jnp.bfloat16)
    gateb = gate.astype(jnp.bfloat16)
    downb = down.astype(jnp.bfloat16)
    dst, wv, meta = _router(flat, router_w)
    te = meta[:, 0]
    tv = meta[:, 1]
    tid, sw = _sc_scatter(dst, wv)
    osh = _shared_ffn(flat, upb[0], gateb[0], downb[0])
    xs = _sc_row_gather(flat, tid, LPR, 11)
    osr = _grouped_ffn(te, tv, sw, xs, upb, gateb, downb)
    yp = _sc_row_gather(osr, dst.reshape(NP), NP, 4)
    out = _combine(osh, yp.reshape(T, 2 * D))
    return out.reshape(orig_shape)


# speedup vs baseline: 1.2608x; 1.2608x over previous
import functools

import jax
import jax.numpy as jnp
from jax import lax
from jax.experimental import pallas as pl
from jax.experimental.pallas import tpu as pltpu
from jax.experimental.pallas import tpu_sc as plsc

E = 8
SHARED = 1
TOPK = 2
D = 1024
DFF = 2048
NR = E - SHARED

T = 2048
M = 256
NTS = T // M
NTR = (TOPK * T) // M + (NR - 1)
NT = NTS + NTR
NTPAD = 32
LPR = NTR * M
NP = TOPK * T


def _router_body(x_ref, rw_ref, dst_ref, wv_ref, meta_ref):
    logits = lax.dot_general(
        x_ref[...], rw_ref[...], (((1,), (1,)), ((), ())),
        preferred_element_type=jnp.float32)
    m = jnp.max(logits, axis=-1, keepdims=True)
    ex = jnp.exp(logits - m)
    probs = ex / jnp.sum(ex, axis=-1, keepdims=True)

    col = lax.broadcasted_iota(jnp.int32, probs.shape, 1)
    v1 = jnp.max(probs, axis=-1, keepdims=True)
    i1 = jnp.min(jnp.where(probs == v1, col, NR), axis=-1, keepdims=True)
    pm = jnp.where(col == i1, -jnp.inf, probs)
    v2 = jnp.max(pm, axis=-1, keepdims=True)
    i2 = jnp.min(jnp.where(pm == v2, col, NR), axis=-1, keepdims=True)
    denom = v1 + v2 + 1e-9
    wv_ref[...] = jnp.concatenate([v1 / denom, v2 / denom], axis=1)

    a0 = (col == i1).astype(jnp.float32)
    a1 = (col == i2).astype(jnp.float32)
    s = a0 + a1
    r_io = lax.broadcasted_iota(jnp.int32, (T, T), 0)
    c_io = lax.broadcasted_iota(jnp.int32, (T, T), 1)
    tri = (c_io < r_io).astype(jnp.float32)
    xc = lax.dot_general(tri, s, (((1,), (0,)), ((), ())),
                         preferred_element_type=jnp.float32)
    counts = jnp.sum(s, axis=0, keepdims=True)

    nt = jnp.floor((counts + (M - 1)) * (1.0 / M))
    e_r = lax.broadcasted_iota(jnp.int32, (NR, NR), 0)
    e_c = lax.broadcasted_iota(jnp.int32, (NR, NR), 1)
    ltri = (e_c <= e_r).astype(jnp.float32)
    cumi = lax.dot_general(nt, ltri, (((1,), (1,)), ((), ())),
                           preferred_element_type=jnp.float32)
    cume = cumi - nt
    pb = M * cume
    dst0 = jnp.sum(a0 * (pb + xc), axis=1, keepdims=True)
    dst1 = jnp.sum(a1 * (pb + xc), axis=1, keepdims=True)
    dst_ref[...] = jnp.concatenate([dst0, dst1], axis=1).astype(jnp.int32)

    j_io = lax.broadcasted_iota(jnp.int32, (NTPAD, NR), 0)
    total_i = jnp.sum(nt, axis=1, keepdims=True).astype(jnp.int32)
    cumi_i = cumi.astype(jnp.int32)
    j_col = lax.broadcasted_iota(jnp.int32, (NTPAD, 1), 0)
    te_full = jnp.minimum(
        jnp.sum((j_io >= cumi_i).astype(jnp.int32), axis=1,
                keepdims=True) + SHARED, E - 1)
    tv_full = (j_col < total_i).astype(jnp.int32)
    meta_ref[...] = jnp.concatenate([te_full, tv_full], axis=1)


def _router(flat, router_w):
    return pl.pallas_call(
        _router_body,
        out_shape=[
            jax.ShapeDtypeStruct((T, 2), jnp.int32),
            jax.ShapeDtypeStruct((T, 2), jnp.float32),
            jax.ShapeDtypeStruct((NTPAD, 2), jnp.int32),
        ],
    )(flat, router_w)


def _sc_info():
    info = plsc.get_sparse_core_info()
    return info.num_cores, info.num_subcores


def _scatter_kernel_fn(dst_hbm, wv_hbm, tid_hbm, sw_hbm,
                       dst_v, wv_v, tid_v, sw_v):
    nc, _ = _sc_info()
    wid = lax.axis_index("s") * nc + lax.axis_index("c")

    @pl.when(wid == 0)
    def _():
        pltpu.sync_copy(dst_hbm, dst_v)
        pltpu.sync_copy(wv_hbm, wv_v)
        lane = lax.iota(jnp.int32, 16)
        half = lax.shift_right_logical(lane, 1)
        zeros_i = jnp.zeros((16,), jnp.int32)
        zeros_f = jnp.zeros((16,), jnp.float32)

        def init_pad(i, carry):
            tid_v[pl.ds(i * 16, 16)] = jnp.bitwise_and(lane + i * 16, T - 1)
            sw_v[pl.ds(i * 16, 16)] = zeros_f
            return carry

        lax.fori_loop(0, LPR // 16, init_pad, 0)

        def scat_body(i, carry):
            base = i * 16
            idx = dst_v[pl.ds(base, 16)]
            vals = wv_v[pl.ds(base, 16)]
            toks = half + i * 8
            plsc.store_scatter(tid_v, [idx], toks)
            plsc.store_scatter(sw_v, [idx], vals)
            return carry

        lax.fori_loop(0, NP // 16, scat_body, 0)
        pltpu.sync_copy(tid_v, tid_hbm)
        pltpu.sync_copy(sw_v, sw_hbm)


def _sc_scatter(dst, wv):
    mesh = plsc.VectorSubcoreMesh(core_axis_name="c", subcore_axis_name="s")
    k = functools.partial(
        pl.kernel, mesh=mesh,
        compiler_params=pltpu.CompilerParams(needs_layout_passes=False),
        out_type=[
            jax.ShapeDtypeStruct((LPR,), jnp.int32),
            jax.ShapeDtypeStruct((LPR,), jnp.float32),
        ],
        scratch_types=[
            pltpu.VMEM((NP,), jnp.int32),
            pltpu.VMEM((NP,), jnp.float32),
            pltpu.VMEM((LPR,), jnp.int32),
            pltpu.VMEM((LPR,), jnp.float32),
        ],
    )(_scatter_kernel_fn)
    return k(dst.reshape(NP), wv.reshape(NP))


def _row_gather_body(src_hbm, idx_hbm, out_hbm, idx_v, rows_a, rows_b, sem,
                     *, per_w, nch):
    nc, ns = _sc_info()
    wid = lax.axis_index("s") * nc + lax.axis_index("c")
    chunk = per_w // nch
    base = wid * per_w
    pltpu.sync_copy(idx_hbm.at[pl.ds(base, per_w)], idx_v)
    bufs = (rows_a, rows_b)
    cps = []
    for c in range(nch):
        if c >= 2:
            cps[c - 2].wait()
            pltpu.sync_copy(bufs[(c - 2) % 2],
                            out_hbm.at[pl.ds(base + (c - 2) * chunk, chunk)])
        cps.append(pltpu.async_copy(
            src_hbm.at[idx_v.at[pl.ds(c * chunk, chunk)]], bufs[c % 2], sem))
    for c in range(max(0, nch - 2), nch):
        cps[c].wait()
        pltpu.sync_copy(bufs[c % 2],
                        out_hbm.at[pl.ds(base + c * chunk, chunk)])


def _sc_row_gather(src, idx, n_out, nch):
    mesh = plsc.VectorSubcoreMesh(core_axis_name="c", subcore_axis_name="s")
    nc, ns = _sc_info()
    per_w = n_out // (nc * ns)
    chunk = per_w // nch
    body = functools.partial(_row_gather_body, per_w=per_w, nch=nch)
    k = functools.partial(
        pl.kernel, mesh=mesh,
        out_type=jax.ShapeDtypeStruct((n_out, D), jnp.float32),
        scratch_types=[
            pltpu.VMEM((per_w,), jnp.int32),
            pltpu.VMEM((chunk, D), jnp.float32),
            pltpu.VMEM((chunk, D), jnp.float32),
            pltpu.SemaphoreType.DMA,
        ],
    )(body)
    return k(src, idx)


def _shared_body(x_ref, up_ref, gate_ref, down_ref, out_ref):
    xb = x_ref[...].astype(jnp.bfloat16)
    u = lax.dot_general(xb, up_ref[...], (((1,), (1,)), ((), ())),
                        preferred_element_type=jnp.float32)
    g = lax.dot_general(xb, gate_ref[...], (((1,), (1,)), ((), ())),
                        preferred_element_type=jnp.float32)
    h = (g * jax.nn.sigmoid(g) * u).astype(jnp.bfloat16)
    out_ref[...] = lax.dot_general(h, down_ref[...], (((1,), (1,)), ((), ())),
                                   preferred_element_type=jnp.float32)


def _shared_ffn(flat, up0, gate0, down0):
    MS = 512
    return pl.pallas_call(
        _shared_body,
        grid=(T // MS,),
        in_specs=[
            pl.BlockSpec((MS, D), lambda t: (t, 0)),
            pl.BlockSpec((DFF, D), lambda t: (0, 0)),
            pl.BlockSpec((DFF, D), lambda t: (0, 0)),
            pl.BlockSpec((D, DFF), lambda t: (0, 0)),
        ],
        out_specs=pl.BlockSpec((MS, D), lambda t: (t, 0)),
        out_shape=jax.ShapeDtypeStruct((T, D), jnp.float32),
    )(flat, up0, gate0, down0)


def _gffn_body(te_ref, tv_ref, sw_ref, xs_ref, up_ref, gate_ref, down_ref,
               os_ref):
    i = pl.program_id(0)

    @pl.when(tv_ref[i] == 1)
    def _():
        xb = xs_ref[...].astype(jnp.bfloat16)
        u = lax.dot_general(xb, up_ref[0], (((1,), (1,)), ((), ())),
                            preferred_element_type=jnp.float32)
        g = lax.dot_general(xb, gate_ref[0], (((1,), (1,)), ((), ())),
                            preferred_element_type=jnp.float32)
        h = g * jax.nn.sigmoid(g) * u
        h = (h * sw_ref[0, 0][:, None]).astype(jnp.bfloat16)
        os_ref[...] = lax.dot_general(h, down_ref[0], (((1,), (1,)), ((), ())),
                                      preferred_element_type=jnp.float32)


def _grouped_ffn(te, tv, sw, xs, up, gate, down):
    grid_spec = pltpu.PrefetchScalarGridSpec(
        num_scalar_prefetch=2,
        grid=(NTR,),
        in_specs=[
            pl.BlockSpec((1, 1, M), lambda i, te, tv: (i, 0, 0)),
            pl.BlockSpec((M, D), lambda i, te, tv: (i, 0)),
            pl.BlockSpec((1, DFF, D), lambda i, te, tv: (te[i], 0, 0)),
            pl.BlockSpec((1, DFF, D), lambda i, te, tv: (te[i], 0, 0)),
            pl.BlockSpec((1, D, DFF), lambda i, te, tv: (te[i], 0, 0)),
        ],
        out_specs=pl.BlockSpec((M, D), lambda i, te, tv: (i, 0)),
    )
    return pl.pallas_call(
        _gffn_body,
        grid_spec=grid_spec,
        out_shape=jax.ShapeDtypeStruct((LPR, D), jnp.float32),
        compiler_params=pltpu.CompilerParams(
            dimension_semantics=("arbitrary",),
        ),
    )(te, tv, sw.reshape(NTR, 1, M), xs, up, gate, down)


def _combine_body(os_ref, yp_ref, out_ref):
    out_ref[...] = (os_ref[...] + yp_ref[:, :D] + yp_ref[:, D:])


def _combine(os, yp2):
    BT = 512
    return pl.pallas_call(
        _combine_body,
        grid=(T // BT,),
        in_specs=[
            pl.BlockSpec((BT, D), lambda t: (t, 0)),
            pl.BlockSpec((BT, 2 * D), lambda t: (t, 0)),
        ],
        out_specs=pl.BlockSpec((BT, D), lambda t: (t, 0)),
        out_shape=jax.ShapeDtypeStruct((T, D), jnp.float32),
    )(os, yp2)


def kernel(x, up, gate, down, router_w):
    orig_shape = x.shape
    flat = x.reshape(-1, D)
    upb = up.astype(jnp.bfloat16)
    gateb = gate.astype(jnp.bfloat16)
    downb = down.astype(jnp.bfloat16)
    dst, wv, meta = _router(flat, router_w)
    te = meta[:, 0]
    tv = meta[:, 1]
    tid, sw = _sc_scatter(dst, wv)
    osh = _shared_ffn(flat, upb[0], gateb[0], downb[0])
    xs = _sc_row_gather(flat, tid, LPR, 11)
    osr = _grouped_ffn(te, tv, sw, xs, upb, gateb, downb)
    yp = _sc_row_gather(osr, dst.reshape(NP), NP, 4)
    out = _combine(osh, yp.reshape(T, 2 * D))
    return out.reshape(orig_shape)
